# baseline (speedup 1.0000x reference)
import functools

import jax
import jax.numpy as jnp
from jax import lax
from jax.experimental import pallas as pl
from jax.experimental.pallas import tpu as pltpu
from jax.experimental.pallas import tpu_sc as plsc

_CHUNK = 128
_NBUF = 4
_TBUF = 2
_LAG = 3


_TC = 128


@functools.lru_cache(maxsize=None)
def _make_transpose(V, D, NC, NS):
    NW = NC * NS
    n_full = V // _TC
    tail = V % _TC
    n_cc = -(-n_full // NW)
    assert n_cc >= 4 and tail % 8 == 0
    mesh = plsc.VectorSubcoreMesh(core_axis_name="c", subcore_axis_name="s")

    @functools.partial(
        pl.kernel,
        mesh=mesh,
        out_type=jax.ShapeDtypeStruct((V, 2 * D), jnp.float32),
        scratch_types=[
            [pltpu.VMEM((D, _TC), jnp.float32) for _ in range(2)],
            [pltpu.VMEM((_TC, 2 * D), jnp.float32) for _ in range(2)],
            [pltpu.SemaphoreType.DMA for _ in range(2)],
            [pltpu.SemaphoreType.DMA for _ in range(2)],
        ],
        compiler_params=pltpu.CompilerParams(
            use_tc_tiling_on_sc=True, needs_layout_passes=False),
    )
    def k(tt_hbm, tail_hbm, out_hbm, ibufs, obufs, rsem, wsem):
        wid = lax.axis_index("s") * NC + lax.axis_index("c")
        base = wid * n_full // NW
        last_c = (wid + 1) * n_full // NW - 1

        def chunk(i):
            return lax.min(base + i, last_c)

        def rd(c, b):
            return pltpu.make_async_copy(
                tt_hbm.at[:, pl.ds(c * _TC, _TC)], ibufs[b], rsem[b])

        def wr(c, b):
            return pltpu.make_async_copy(
                obufs[b], out_hbm.at[pl.ds(c * _TC, _TC), :], wsem[b])

        def transp(b):
            rows = lax.iota(jnp.int32, 16)
            zero16 = jnp.zeros((16,), jnp.int32)

            @plsc.parallel_loop(0, _TC, unroll=8)
            def ubody(u):
                cols = zero16 + u
                for g in range(D // 16):
                    vals = plsc.load_gather(ibufs[b], [rows + 16 * g, cols])
                    obufs[b][u, pl.ds(16 * g, 16)] = vals

        def cstep(i, b, first, no_next_read):
            rd(chunk(i), b).wait()
            if not first:
                wr(chunk(i - 2), b).wait()
            transp(b)
            wr(chunk(i), b).start()
            if not no_next_read:
                rd(chunk(i + 2), b).start()

        rd(chunk(0), 0).start()
        rd(chunk(1), 1).start()
        for b in range(2):
            cstep(b, b, True, False)

        def group(g, carry):
            for b in range(2):
                cstep(2 * g + b, b, False, False)
            return carry

        lax.fori_loop(1, n_cc // 2 - 1, group, 0)

        i0 = 2 * (n_cc // 2 - 1)
        for i in range(i0, n_cc):
            cstep(i, i % 2, False, i + 2 >= n_cc)
        for i in range(n_cc - 2, n_cc):
            wr(chunk(i), i % 2).wait()

        if tail:
            @pl.when(wid == NW - 1)
            def _():
                pltpu.sync_copy(tail_hbm, obufs[0].at[pl.ds(0, tail), :])
                pltpu.sync_copy(obufs[0].at[pl.ds(0, tail), :],
                                out_hbm.at[pl.ds(V - tail, tail), :])

    return k


@functools.lru_cache(maxsize=None)
def _make_gather(L_, B_, V, D, NC, NS):
    NW = NC * NS
    bw = B_ // NW
    cpl = bw // _CHUNK
    n_chunks = L_ * cpl
    n_groups = n_chunks // _NBUF
    assert n_chunks % _NBUF == 0 and n_groups >= 2
    mesh = plsc.VectorSubcoreMesh(core_axis_name="c", subcore_axis_name="s")

    @functools.partial(
        pl.kernel,
        mesh=mesh,
        out_type=jax.ShapeDtypeStruct((L_, D, B_), jnp.float32),
        scratch_types=[
            pltpu.VMEM((L_, bw), jnp.int32),
            [pltpu.VMEM((_CHUNK, 2 * D), jnp.float32) for _ in range(_NBUF)],
            [pltpu.VMEM((D, _CHUNK), jnp.float32) for _ in range(_TBUF)],
            [pltpu.SemaphoreType.DMA for _ in range(_NBUF)],
            [pltpu.SemaphoreType.DMA for _ in range(_TBUF)],
        ],
        compiler_params=pltpu.CompilerParams(
            use_tc_tiling_on_sc=True, needs_layout_passes=False),
    )
    def k(tab_hbm, xt_hbm, out_hbm, idx_v, gbufs, tbufs, gsem, wsem):
        wid = lax.axis_index("s") * NC + lax.axis_index("c")
        b0 = wid * bw
        pltpu.sync_copy(xt_hbm.at[:, pl.ds(b0, bw)], idx_v)

        def gather(i, gb):
            l, j = i // cpl, i % cpl
            return pltpu.make_async_copy(
                tab_hbm.at[idx_v.at[l, pl.ds(j * _CHUNK, _CHUNK)]],
                gbufs[gb], gsem[gb])

        def write(i, tb):
            l, j = i // cpl, i % cpl
            return pltpu.make_async_copy(
                tbufs[tb],
                out_hbm.at[l, :, pl.ds(b0 + j * _CHUNK, _CHUNK)], wsem[tb])

        def transpose(gb, tb):
            rows = lax.iota(jnp.int32, 16)
            zero16 = jnp.zeros((16,), jnp.int32)

            @plsc.parallel_loop(0, D, unroll=8)
            def dbody(d):
                cols = zero16 + d
                for g in range(_CHUNK // 16):
                    vals = plsc.load_gather(gbufs[gb], [rows + 16 * g, cols])
                    tbufs[tb][d, pl.ds(16 * g, 16)] = vals

        def step(i, b, first_group, last_group):
            gather(i, b % _NBUF).wait()
            if not (first_group and b < _TBUF):
                write(i - _TBUF, b % _TBUF).wait()
            transpose(b % _NBUF, b % _TBUF)
            write(i, b % _TBUF).start()
            if not (last_group and b >= _NBUF - _LAG):
                gather(i + _LAG, (b + _LAG) % _NBUF).start()

        for b in range(_LAG):
            gather(b, b).start()
        for b in range(_NBUF):
            step(b, b, True, False)

        def group(g, carry):
            for b in range(_NBUF):
                step(g * _NBUF + b, b, False, False)
            return carry

        lax.fori_loop(1, n_groups - 1, group, 0)

        i0 = (n_groups - 1) * _NBUF
        for b in range(_NBUF):
            step(i0 + b, b, False, True)
        for b in range(_TBUF):
            write(n_chunks - _TBUF + b, (n_chunks - _TBUF + b) % _TBUF).wait()

    return k


def kernel(x, table):
    B_, L_ = x.shape
    V, D = table.shape
    info = plsc.get_sparse_core_info()
    NC, NS = info.num_cores, info.num_subcores
    tail = V % _TC
    tail_p = jnp.pad(table[V - tail:, :], ((0, 0), (0, D)))
    table_p = _make_transpose(V, D, NC, NS)(table.T, tail_p)
    xt = x.T.astype(jnp.int32)
    out_t = _make_gather(L_, B_, V, D, NC, NS)(table_p, xt)
    return out_t.transpose(2, 0, 1)


# speedup vs baseline: 1.0005x; 1.0005x over previous
import functools

import jax
import jax.numpy as jnp
from jax import lax
from jax.experimental import pallas as pl
from jax.experimental.pallas import tpu as pltpu
from jax.experimental.pallas import tpu_sc as plsc

_CHUNK = 128
_NBUF = 4
_TBUF = 2
_LAG = 3


_TC = 256


@functools.lru_cache(maxsize=None)
def _make_transpose(V, D, NC, NS):
    NW = NC * NS
    n_full = V // _TC
    tail = V % _TC
    n_cc = -(-n_full // NW)
    assert n_cc >= 4 and tail % 8 == 0
    mesh = plsc.VectorSubcoreMesh(core_axis_name="c", subcore_axis_name="s")

    @functools.partial(
        pl.kernel,
        mesh=mesh,
        out_type=jax.ShapeDtypeStruct((V, 2 * D), jnp.float32),
        scratch_types=[
            [pltpu.VMEM((D, _TC), jnp.float32) for _ in range(2)],
            [pltpu.VMEM((_TC, 2 * D), jnp.float32) for _ in range(2)],
            [pltpu.SemaphoreType.DMA for _ in range(2)],
            [pltpu.SemaphoreType.DMA for _ in range(2)],
        ],
        compiler_params=pltpu.CompilerParams(
            use_tc_tiling_on_sc=True, needs_layout_passes=False),
    )
    def k(tt_hbm, tail_hbm, out_hbm, ibufs, obufs, rsem, wsem):
        wid = lax.axis_index("s") * NC + lax.axis_index("c")
        base = wid * n_full // NW
        last_c = (wid + 1) * n_full // NW - 1

        def chunk(i):
            return lax.min(base + i, last_c)

        def rd(c, b):
            return pltpu.make_async_copy(
                tt_hbm.at[:, pl.ds(c * _TC, _TC)], ibufs[b], rsem[b])

        def wr(c, b):
            return pltpu.make_async_copy(
                obufs[b], out_hbm.at[pl.ds(c * _TC, _TC), :], wsem[b])

        def transp(b):
            rows = lax.iota(jnp.int32, 16)
            zero16 = jnp.zeros((16,), jnp.int32)

            @plsc.parallel_loop(0, _TC, unroll=8)
            def ubody(u):
                cols = zero16 + u
                for g in range(D // 16):
                    vals = plsc.load_gather(ibufs[b], [rows + 16 * g, cols])
                    obufs[b][u, pl.ds(16 * g, 16)] = vals

        def cstep(i, b, first, no_next_read):
            rd(chunk(i), b).wait()
            if not first:
                wr(chunk(i - 2), b).wait()
            transp(b)
            wr(chunk(i), b).start()
            if not no_next_read:
                rd(chunk(i + 2), b).start()

        rd(chunk(0), 0).start()
        rd(chunk(1), 1).start()
        for b in range(2):
            cstep(b, b, True, False)

        def group(g, carry):
            for b in range(2):
                cstep(2 * g + b, b, False, False)
            return carry

        lax.fori_loop(1, n_cc // 2 - 1, group, 0)

        i0 = 2 * (n_cc // 2 - 1)
        for i in range(i0, n_cc):
            cstep(i, i % 2, False, i + 2 >= n_cc)
        for i in range(n_cc - 2, n_cc):
            wr(chunk(i), i % 2).wait()

        if tail:
            @pl.when(wid == NW - 1)
            def _():
                pltpu.sync_copy(tail_hbm, obufs[0].at[pl.ds(0, tail), :])
                pltpu.sync_copy(obufs[0].at[pl.ds(0, tail), :],
                                out_hbm.at[pl.ds(V - tail, tail), :])

    return k


@functools.lru_cache(maxsize=None)
def _make_gather(L_, B_, V, D, NC, NS):
    NW = NC * NS
    bw = B_ // NW
    cpl = bw // _CHUNK
    n_chunks = L_ * cpl
    n_groups = n_chunks // _NBUF
    assert n_chunks % _NBUF == 0 and n_groups >= 2
    mesh = plsc.VectorSubcoreMesh(core_axis_name="c", subcore_axis_name="s")

    @functools.partial(
        pl.kernel,
        mesh=mesh,
        out_type=jax.ShapeDtypeStruct((L_, D, B_), jnp.float32),
        scratch_types=[
            pltpu.VMEM((L_, bw), jnp.int32),
            [pltpu.VMEM((_CHUNK, 2 * D), jnp.float32) for _ in range(_NBUF)],
            [pltpu.VMEM((D, _CHUNK), jnp.float32) for _ in range(_TBUF)],
            [pltpu.SemaphoreType.DMA for _ in range(_NBUF)],
            [pltpu.SemaphoreType.DMA for _ in range(_TBUF)],
        ],
        compiler_params=pltpu.CompilerParams(
            use_tc_tiling_on_sc=True, needs_layout_passes=False),
    )
    def k(tab_hbm, xt_hbm, out_hbm, idx_v, gbufs, tbufs, gsem, wsem):
        wid = lax.axis_index("s") * NC + lax.axis_index("c")
        b0 = wid * bw
        pltpu.sync_copy(xt_hbm.at[:, pl.ds(b0, bw)], idx_v)

        def gather(i, gb):
            l, j = i // cpl, i % cpl
            return pltpu.make_async_copy(
                tab_hbm.at[idx_v.at[l, pl.ds(j * _CHUNK, _CHUNK)]],
                gbufs[gb], gsem[gb])

        def write(i, tb):
            l, j = i // cpl, i % cpl
            return pltpu.make_async_copy(
                tbufs[tb],
                out_hbm.at[l, :, pl.ds(b0 + j * _CHUNK, _CHUNK)], wsem[tb])

        def transpose(gb, tb):
            rows = lax.iota(jnp.int32, 16)
            zero16 = jnp.zeros((16,), jnp.int32)

            @plsc.parallel_loop(0, D, unroll=8)
            def dbody(d):
                cols = zero16 + d
                for g in range(_CHUNK // 16):
                    vals = plsc.load_gather(gbufs[gb], [rows + 16 * g, cols])
                    tbufs[tb][d, pl.ds(16 * g, 16)] = vals

        def step(i, b, first_group, last_group):
            gather(i, b % _NBUF).wait()
            if not (first_group and b < _TBUF):
                write(i - _TBUF, b % _TBUF).wait()
            transpose(b % _NBUF, b % _TBUF)
            write(i, b % _TBUF).start()
            if not (last_group and b >= _NBUF - _LAG):
                gather(i + _LAG, (b + _LAG) % _NBUF).start()

        for b in range(_LAG):
            gather(b, b).start()
        for b in range(_NBUF):
            step(b, b, True, False)

        def group(g, carry):
            for b in range(_NBUF):
                step(g * _NBUF + b, b, False, False)
            return carry

        lax.fori_loop(1, n_groups - 1, group, 0)

        i0 = (n_groups - 1) * _NBUF
        for b in range(_NBUF):
            step(i0 + b, b, False, True)
        for b in range(_TBUF):
            write(n_chunks - _TBUF + b, (n_chunks - _TBUF + b) % _TBUF).wait()

    return k


def kernel(x, table):
    B_, L_ = x.shape
    V, D = table.shape
    info = plsc.get_sparse_core_info()
    NC, NS = info.num_cores, info.num_subcores
    tail = V % _TC
    tail_p = jnp.pad(table[V - tail:, :], ((0, 0), (0, D)))
    table_p = _make_transpose(V, D, NC, NS)(table.T, tail_p)
    xt = x.T.astype(jnp.int32)
    out_t = _make_gather(L_, B_, V, D, NC, NS)(table_p, xt)
    return out_t.transpose(2, 0, 1)


# speedup vs baseline: 1.2978x; 1.2971x over previous
import functools

import jax
import jax.numpy as jnp
from jax import lax
from jax.experimental import pallas as pl
from jax.experimental.pallas import tpu as pltpu
from jax.experimental.pallas import tpu_sc as plsc

_CHUNK = 128
_NBUF = 4
_TBUF = 2
_LAG = 3


_TC = 256


@functools.lru_cache(maxsize=None)
def _make_transpose(V, D, NC, NS):
    NW = NC * NS
    n_full = V // _TC
    tail = V % _TC
    n_cc = -(-n_full // NW)
    assert n_cc >= 4 and tail % 8 == 0
    mesh = plsc.VectorSubcoreMesh(core_axis_name="c", subcore_axis_name="s")

    @functools.partial(
        pl.kernel,
        mesh=mesh,
        out_type=jax.ShapeDtypeStruct((V, 2 * D), jnp.float32),
        scratch_types=[
            [pltpu.VMEM((D, _TC), jnp.float32) for _ in range(2)],
            [pltpu.VMEM((_TC, 2 * D + 1), jnp.float32) for _ in range(2)],
            [pltpu.SemaphoreType.DMA for _ in range(2)],
            [pltpu.SemaphoreType.DMA for _ in range(2)],
        ],
        compiler_params=pltpu.CompilerParams(
            use_tc_tiling_on_sc=True, needs_layout_passes=False),
    )
    def k(tt_hbm, tail_hbm, out_hbm, ibufs, obufs, rsem, wsem):
        wid = lax.axis_index("s") * NC + lax.axis_index("c")
        base = wid * n_full // NW
        last_c = (wid + 1) * n_full // NW - 1

        def chunk(i):
            return lax.min(base + i, last_c)

        def rd(c, b):
            return pltpu.make_async_copy(
                tt_hbm.at[:, pl.ds(c * _TC, _TC)], ibufs[b], rsem[b])

        def wr(c, b):
            return pltpu.make_async_copy(
                obufs[b].at[:, pl.ds(0, 2 * D)],
                out_hbm.at[pl.ds(c * _TC, _TC), :], wsem[b])

        def transp(b):
            rows = lax.iota(jnp.int32, 16)
            zero16 = jnp.zeros((16,), jnp.int32)

            @plsc.parallel_loop(0, D, unroll=8)
            def dbody(d):
                cols = zero16 + d
                for g in range(_TC // 16):
                    vals = ibufs[b][d, pl.ds(16 * g, 16)]
                    plsc.store_scatter(obufs[b], [rows + 16 * g, cols], vals)

        def cstep(i, b, first, no_next_read):
            rd(chunk(i), b).wait()
            if not first:
                wr(chunk(i - 2), b).wait()
            transp(b)
            wr(chunk(i), b).start()
            if not no_next_read:
                rd(chunk(i + 2), b).start()

        rd(chunk(0), 0).start()
        rd(chunk(1), 1).start()
        for b in range(2):
            cstep(b, b, True, False)

        def group(g, carry):
            for b in range(2):
                cstep(2 * g + b, b, False, False)
            return carry

        lax.fori_loop(1, n_cc // 2 - 1, group, 0)

        i0 = 2 * (n_cc // 2 - 1)
        for i in range(i0, n_cc):
            cstep(i, i % 2, False, i + 2 >= n_cc)
        for i in range(n_cc - 2, n_cc):
            wr(chunk(i), i % 2).wait()

        if tail:
            @pl.when(wid == NW - 1)
            def _():
                pltpu.sync_copy(tail_hbm,
                                obufs[0].at[pl.ds(0, tail), pl.ds(0, 2 * D)])
                pltpu.sync_copy(obufs[0].at[pl.ds(0, tail), pl.ds(0, 2 * D)],
                                out_hbm.at[pl.ds(V - tail, tail), :])

    return k


@functools.lru_cache(maxsize=None)
def _make_gather(L_, B_, V, D, NC, NS):
    NW = NC * NS
    bw = B_ // NW
    cpl = bw // _CHUNK
    n_chunks = L_ * cpl
    n_groups = n_chunks // _NBUF
    assert n_chunks % _NBUF == 0 and n_groups >= 2
    mesh = plsc.VectorSubcoreMesh(core_axis_name="c", subcore_axis_name="s")

    @functools.partial(
        pl.kernel,
        mesh=mesh,
        out_type=jax.ShapeDtypeStruct((L_, D, B_), jnp.float32),
        scratch_types=[
            pltpu.VMEM((L_, bw), jnp.int32),
            [pltpu.VMEM((_CHUNK, 2 * D), jnp.float32) for _ in range(_NBUF)],
            [pltpu.VMEM((D, _CHUNK), jnp.float32) for _ in range(_TBUF)],
            [pltpu.SemaphoreType.DMA for _ in range(_NBUF)],
            [pltpu.SemaphoreType.DMA for _ in range(_TBUF)],
        ],
        compiler_params=pltpu.CompilerParams(
            use_tc_tiling_on_sc=True, needs_layout_passes=False),
    )
    def k(tab_hbm, xt_hbm, out_hbm, idx_v, gbufs, tbufs, gsem, wsem):
        wid = lax.axis_index("s") * NC + lax.axis_index("c")
        b0 = wid * bw
        pltpu.sync_copy(xt_hbm.at[:, pl.ds(b0, bw)], idx_v)

        def gather(i, gb):
            l, j = i // cpl, i % cpl
            return pltpu.make_async_copy(
                tab_hbm.at[idx_v.at[l, pl.ds(j * _CHUNK, _CHUNK)]],
                gbufs[gb], gsem[gb])

        def write(i, tb):
            l, j = i // cpl, i % cpl
            return pltpu.make_async_copy(
                tbufs[tb],
                out_hbm.at[l, :, pl.ds(b0 + j * _CHUNK, _CHUNK)], wsem[tb])

        def transpose(gb, tb):
            rows = lax.iota(jnp.int32, 16)
            zero16 = jnp.zeros((16,), jnp.int32)

            @plsc.parallel_loop(0, _CHUNK, unroll=8)
            def jbody(j):
                cols = zero16 + j
                for g in range(D // 16):
                    vals = gbufs[gb][j, pl.ds(16 * g, 16)]
                    plsc.store_scatter(tbufs[tb], [rows + 16 * g, cols], vals)

        def step(i, b, first_group, last_group):
            gather(i, b % _NBUF).wait()
            if not (first_group and b < _TBUF):
                write(i - _TBUF, b % _TBUF).wait()
            transpose(b % _NBUF, b % _TBUF)
            write(i, b % _TBUF).start()
            if not (last_group and b >= _NBUF - _LAG):
                gather(i + _LAG, (b + _LAG) % _NBUF).start()

        for b in range(_LAG):
            gather(b, b).start()
        for b in range(_NBUF):
            step(b, b, True, False)

        def group(g, carry):
            for b in range(_NBUF):
                step(g * _NBUF + b, b, False, False)
            return carry

        lax.fori_loop(1, n_groups - 1, group, 0)

        i0 = (n_groups - 1) * _NBUF
        for b in range(_NBUF):
            step(i0 + b, b, False, True)
        for b in range(_TBUF):
            write(n_chunks - _TBUF + b, (n_chunks - _TBUF + b) % _TBUF).wait()

    return k


def kernel(x, table):
    B_, L_ = x.shape
    V, D = table.shape
    info = plsc.get_sparse_core_info()
    NC, NS = info.num_cores, info.num_subcores
    table_p = jnp.pad(table, ((0, 0), (0, D)))
    xt = x.T.astype(jnp.int32)
    out_t = _make_gather(L_, B_, V, D, NC, NS)(table_p, xt)
    return out_t.transpose(2, 0, 1)


# speedup vs baseline: 1.3316x; 1.0261x over previous
import functools

import jax
import jax.numpy as jnp
from jax import lax
from jax.experimental import pallas as pl
from jax.experimental.pallas import tpu as pltpu
from jax.experimental.pallas import tpu_sc as plsc

_CHUNK = 128
_NBUF = 4
_TBUF = 2
_LAG = 3


@functools.lru_cache(maxsize=None)
def _make_gather(L_, B_, V, D, NC, NS):
    NW = NC * NS
    bw = B_ // NW
    cpl = bw // _CHUNK
    n_chunks = L_ * cpl
    n_groups = n_chunks // _NBUF
    assert n_chunks % _NBUF == 0 and n_groups >= 2
    mesh = plsc.VectorSubcoreMesh(core_axis_name="c", subcore_axis_name="s")

    @functools.partial(
        pl.kernel,
        mesh=mesh,
        out_type=jax.ShapeDtypeStruct((L_, D, B_), jnp.float32),
        scratch_types=[
            pltpu.VMEM((L_, bw), jnp.int32),
            [pltpu.VMEM((_CHUNK, 2 * D), jnp.float32) for _ in range(_NBUF)],
            [pltpu.VMEM((D, _CHUNK), jnp.float32) for _ in range(_TBUF)],
            [pltpu.SemaphoreType.DMA for _ in range(_NBUF)],
            [pltpu.SemaphoreType.DMA for _ in range(_TBUF)],
        ],
        compiler_params=pltpu.CompilerParams(
            use_tc_tiling_on_sc=True, needs_layout_passes=False),
    )
    def k(tab_hbm, xt_hbm, out_hbm, idx_v, gbufs, tbufs, gsem, wsem):
        wid = lax.axis_index("s") * NC + lax.axis_index("c")
        b0 = wid * bw
        pltpu.sync_copy(xt_hbm.at[:, pl.ds(b0, bw)], idx_v)

        def gather(i, gb):
            l, j = i // cpl, i % cpl
            return pltpu.make_async_copy(
                tab_hbm.at[idx_v.at[l, pl.ds(j * _CHUNK, _CHUNK)]],
                gbufs[gb], gsem[gb])

        def write(i, tb):
            l, j = i // cpl, i % cpl
            return pltpu.make_async_copy(
                tbufs[tb],
                out_hbm.at[l, :, pl.ds(b0 + j * _CHUNK, _CHUNK)], wsem[tb])

        def transpose(gb, tb):
            rows = lax.iota(jnp.int32, 16)
            zero16 = jnp.zeros((16,), jnp.int32)

            @plsc.parallel_loop(0, D, unroll=8)
            def dbody(d):
                cols = zero16 + d
                for g in range(_CHUNK // 16):
                    vals = plsc.load_gather(gbufs[gb], [rows + 16 * g, cols])
                    tbufs[tb][d, pl.ds(16 * g, 16)] = vals

        def step(i, b, first_group, last_group):
            gather(i, b % _NBUF).wait()
            if not (first_group and b < _TBUF):
                write(i - _TBUF, b % _TBUF).wait()
            transpose(b % _NBUF, b % _TBUF)
            write(i, b % _TBUF).start()
            if not (last_group and b >= _NBUF - _LAG):
                gather(i + _LAG, (b + _LAG) % _NBUF).start()

        for b in range(_LAG):
            gather(b, b).start()
        for b in range(_NBUF):
            step(b, b, True, False)

        def group(g, carry):
            for b in range(_NBUF):
                step(g * _NBUF + b, b, False, False)
            return carry

        lax.fori_loop(1, n_groups - 1, group, 0)

        i0 = (n_groups - 1) * _NBUF
        for b in range(_NBUF):
            step(i0 + b, b, False, True)
        for b in range(_TBUF):
            write(n_chunks - _TBUF + b, (n_chunks - _TBUF + b) % _TBUF).wait()

    return k


def kernel(x, table):
    B_, L_ = x.shape
    V, D = table.shape
    info = plsc.get_sparse_core_info()
    NC, NS = info.num_cores, info.num_subcores
    table_p = jnp.pad(table, ((0, 0), (0, D)))
    xt = x.T.astype(jnp.int32)
    out_t = _make_gather(L_, B_, V, D, NC, NS)(table_p, xt)
    return out_t.transpose(2, 0, 1)
